# Initial kernel scaffold; baseline (speedup 1.0000x reference)
import jax
import jax.numpy as jnp
from jax.experimental import pallas as pl


def kernel(x):
    raise NotImplementedError("write your pallas kernel here")


# speedup vs baseline: 6.7316x; 6.7316x over previous
import jax
import jax.numpy as jnp
from jax.experimental import pallas as pl
from jax.experimental.pallas import tpu as pltpu


def _norm_select_kernel(x_ref, kx_ref, norms_ref, *, nb, nc, d, k):
    c = pl.program_id(1)
    xb = x_ref[0]
    sq = xb * xb
    n64 = sq[:, 0:d] + sq[:, d : 2 * d] + sq[:, 2 * d : 3 * d]
    norms_ref[pl.ds(c * nb, nb), :] = n64

    @pl.when(c == nc - 1)
    def _():
        u = jax.lax.bitcast_convert_type(norms_ref[...], jnp.int32)

        def body(_, carry):
            lo, hi = carry
            mid = jax.lax.shift_right_logical(lo + hi, 1)
            cnt = jnp.sum((u <= mid).astype(jnp.int32), axis=0, keepdims=True)
            pred = cnt >= k
            lo2 = jnp.where(pred, lo, mid + 1)
            hi2 = jnp.where(pred, mid, hi)
            return (lo2, hi2)

        lo0 = jnp.zeros((1, d), jnp.int32)
        hi0 = jnp.full((1, d), jnp.int32(0x7FFFFFFF))
        lo, hi = jax.lax.fori_loop(0, 31, body, (lo0, hi0))
        kx_ref[0] = jax.lax.bitcast_convert_type(lo, jnp.float32)


def _mask_kernel(x_ref, kx_ref, o_ref, *, d):
    xb = x_ref[0]
    sq = xb * xb
    n64 = sq[:, 0:d] + sq[:, d : 2 * d] + sq[:, 2 * d : 3 * d]
    m = (n64 > kx_ref[0]).astype(jnp.float32)
    m3 = jnp.concatenate([m, m, m], axis=-1)
    o_ref[0] = xb * m3


def kernel(x):
    b, n, c3, d = x.shape
    assert c3 == 3
    k = n // 10
    nb = 2048
    nc = n // nb
    l3 = c3 * d

    xr = x.reshape(b, n, l3)

    import functools

    kx = pl.pallas_call(
        functools.partial(_norm_select_kernel, nb=nb, nc=nc, d=d, k=k),
        grid=(b, nc),
        in_specs=[pl.BlockSpec((1, nb, l3), lambda bi, ci: (bi, ci, 0))],
        out_specs=pl.BlockSpec((1, 1, d), lambda bi, ci: (bi, 0, 0)),
        out_shape=jax.ShapeDtypeStruct((b, 1, d), jnp.float32),
        scratch_shapes=[pltpu.VMEM((n, d), jnp.float32)],
    )(xr)

    out = pl.pallas_call(
        functools.partial(_mask_kernel, d=d),
        grid=(b, nc),
        in_specs=[
            pl.BlockSpec((1, nb, l3), lambda bi, ci: (bi, ci, 0)),
            pl.BlockSpec((1, 1, d), lambda bi, ci: (bi, 0, 0)),
        ],
        out_specs=pl.BlockSpec((1, nb, l3), lambda bi, ci: (bi, ci, 0)),
        out_shape=jax.ShapeDtypeStruct((b, n, l3), jnp.float32),
    )(xr, kx)

    return out.reshape(b, n, c3, d)
